# Initial kernel scaffold; baseline (speedup 1.0000x reference)
import jax
import jax.numpy as jnp
from jax.experimental import pallas as pl


def kernel(x, token_emb, pos_emb):
    raise NotImplementedError("write your pallas kernel here")


# speedup vs baseline: 3.8373x; 3.8373x over previous
import functools

import jax
import jax.numpy as jnp
from jax import lax
from jax.experimental import pallas as pl
from jax.experimental.pallas import tpu as pltpu
from jax.experimental.pallas import tpu_sc as plsc


def kernel(x, token_emb, pos_emb):
    B, L = x.shape
    V, D = token_emb.shape
    N = B * L
    info = plsc.get_sparse_core_info()
    NC, NS = info.num_cores, info.num_subcores
    NW = NC * NS
    seq_per_w = B // NW
    tok_per_w = seq_per_w * L
    g1 = 128
    g2 = L - g1

    x_flat = x.reshape(N)
    mesh = plsc.VectorSubcoreMesh(core_axis_name="c", subcore_axis_name="s")

    @functools.partial(
        pl.kernel,
        mesh=mesh,
        out_type=jax.ShapeDtypeStruct((N, D), jnp.float32),
        compiler_params=pltpu.CompilerParams(use_tc_tiling_on_sc=False),
        scratch_types=[
            pltpu.VMEM((tok_per_w,), jnp.int32),
            pltpu.VMEM((L, D), jnp.float32),
            pltpu.VMEM((L, D), jnp.float32),
            pltpu.SemaphoreType.DMA,
        ],
    )
    def run(x_hbm, tok_hbm, pos_hbm, out_hbm, idx_v, pos_v, rows_v, sem):
        wid = lax.axis_index("s") * NC + lax.axis_index("c")
        base = wid * tok_per_w
        pltpu.sync_copy(x_hbm.at[pl.ds(base, tok_per_w)], idx_v)
        pltpu.sync_copy(pos_hbm, pos_v)

        def seq_body(s, carry):
            off = s * L
            c1 = pltpu.async_copy(
                tok_hbm.at[idx_v.at[pl.ds(off, g1)]],
                rows_v.at[pl.ds(0, g1)], sem)
            c2 = pltpu.async_copy(
                tok_hbm.at[idx_v.at[pl.ds(off + g1, g2)]],
                rows_v.at[pl.ds(g1, g2)], sem)
            c1.wait()
            c2.wait()

            def add_body(j, c):
                rows_v[j, pl.ds(0, 16)] = rows_v[j, pl.ds(0, 16)] + pos_v[j, pl.ds(0, 16)]
                rows_v[j, pl.ds(16, 16)] = rows_v[j, pl.ds(16, 16)] + pos_v[j, pl.ds(16, 16)]
                return c

            lax.fori_loop(0, L, add_body, 0)
            pltpu.sync_copy(rows_v, out_hbm.at[pl.ds(base + off, L)])
            return carry

        lax.fori_loop(0, seq_per_w, seq_body, 0)

    out = run(x_flat, token_emb, pos_emb)
    return out.reshape(B, L, D)


# speedup vs baseline: 4.3338x; 1.1294x over previous
import functools

import jax
import jax.numpy as jnp
from jax import lax
from jax.experimental import pallas as pl
from jax.experimental.pallas import tpu as pltpu
from jax.experimental.pallas import tpu_sc as plsc


def kernel(x, token_emb, pos_emb):
    B, L = x.shape
    V, D = token_emb.shape
    N = B * L
    info = plsc.get_sparse_core_info()
    NC, NS = info.num_cores, info.num_subcores
    NW = NC * NS
    seq_per_w = B // NW
    tok_per_w = seq_per_w * L
    g1 = 128
    g2 = L - g1

    x_flat = x.reshape(N)
    mesh = plsc.VectorSubcoreMesh(core_axis_name="c", subcore_axis_name="s")

    @functools.partial(
        pl.kernel,
        mesh=mesh,
        out_type=jax.ShapeDtypeStruct((N, D), jnp.float32),
        compiler_params=pltpu.CompilerParams(use_tc_tiling_on_sc=False),
        scratch_types=[
            pltpu.VMEM((tok_per_w,), jnp.int32),
            pltpu.VMEM_SHARED((L, D), jnp.float32),
            pltpu.VMEM((L, D), jnp.float32),
            pltpu.SemaphoreType.DMA,
        ],
    )
    def run(x_hbm, tok_hbm, pos_hbm, out_hbm, idx_v, pos_sh, rows_v, sem):
        sid = lax.axis_index("s")
        wid = sid * NC + lax.axis_index("c")
        base = wid * tok_per_w
        pltpu.sync_copy(x_hbm.at[pl.ds(base, tok_per_w)], idx_v)

        @pl.when(sid == 0)
        def _():
            pltpu.sync_copy(pos_hbm, pos_sh)

        plsc.subcore_barrier()

        def seq_body(s, carry):
            off = s * L
            pltpu.sync_copy(pos_sh, rows_v)
            c1 = pltpu.async_copy(
                tok_hbm.at[idx_v.at[pl.ds(off, g1)]],
                rows_v.at[pl.ds(0, g1)], sem, add=True)
            c2 = pltpu.async_copy(
                tok_hbm.at[idx_v.at[pl.ds(off + g1, g2)]],
                rows_v.at[pl.ds(g1, g2)], sem, add=True)
            c1.wait()
            c2.wait()
            pltpu.sync_copy(rows_v, out_hbm.at[pl.ds(base + off, L)])
            return carry

        lax.fori_loop(0, seq_per_w, seq_body, 0)

    out = run(x_flat, token_emb, pos_emb)
    return out.reshape(B, L, D)


# speedup vs baseline: 5.1722x; 1.1935x over previous
import functools

import jax
import jax.numpy as jnp
from jax import lax
from jax.experimental import pallas as pl
from jax.experimental.pallas import tpu as pltpu
from jax.experimental.pallas import tpu_sc as plsc

NBUF = 8


def kernel(x, token_emb, pos_emb):
    B, L = x.shape
    V, D = token_emb.shape
    N = B * L
    info = plsc.get_sparse_core_info()
    NC, NS = info.num_cores, info.num_subcores
    NW = NC * NS
    S = B // NW
    tok_per_w = S * L
    g1 = 128
    g2 = L - g1

    x_flat = x.reshape(N)
    mesh = plsc.VectorSubcoreMesh(core_axis_name="c", subcore_axis_name="s")

    @functools.partial(
        pl.kernel,
        mesh=mesh,
        out_type=jax.ShapeDtypeStruct((N, D), jnp.float32),
        compiler_params=pltpu.CompilerParams(use_tc_tiling_on_sc=False),
        scratch_types=[
            pltpu.VMEM((tok_per_w,), jnp.int32),
            pltpu.VMEM_SHARED((L, D), jnp.float32),
            pltpu.VMEM((NBUF, L, D), jnp.float32),
            pltpu.SemaphoreType.DMA((NBUF,)),
            pltpu.SemaphoreType.DMA((NBUF,)),
            pltpu.SemaphoreType.DMA((NBUF,)),
        ],
    )
    def run(x_hbm, tok_hbm, pos_hbm, out_hbm, idx_v, pos_sh, rows_v,
            tsem, gsem, ssem):
        sid = lax.axis_index("s")
        wid = sid * NC + lax.axis_index("c")
        base = wid * tok_per_w
        pltpu.sync_copy(x_hbm.at[pl.ds(base, tok_per_w)], idx_v)

        @pl.when(sid == 0)
        def _():
            pltpu.sync_copy(pos_hbm, pos_sh)

        plsc.subcore_barrier()

        def t_issue(b):
            pltpu.async_copy(pos_sh, rows_v.at[b], tsem.at[b])

        def t_wait(b):
            pltpu.make_async_copy(pos_sh, rows_v.at[b], tsem.at[b]).wait()

        def g_issue(m, b):
            off = m * L
            pltpu.async_copy(
                tok_hbm.at[idx_v.at[pl.ds(off, g1)]],
                rows_v.at[b, pl.ds(0, g1)], gsem.at[b], add=True)
            pltpu.async_copy(
                tok_hbm.at[idx_v.at[pl.ds(off + g1, g2)]],
                rows_v.at[b, pl.ds(g1, g2)], gsem.at[b], add=True)

        def g_wait(b):
            pltpu.make_async_copy(out_hbm.at[pl.ds(base, L)],
                                  rows_v.at[b], gsem.at[b]).wait()

        def s_issue(m, b):
            pltpu.async_copy(rows_v.at[b],
                             out_hbm.at[pl.ds(base + m * L, L)], ssem.at[b])

        def s_wait(b):
            pltpu.make_async_copy(rows_v.at[b],
                                  out_hbm.at[pl.ds(base, L)], ssem.at[b]).wait()

        t_issue(0)
        t_issue(1)
        t_wait(0)
        g_issue(0, 0)

        def outer(s0, carry):
            for j in range(NBUF):
                m = s0 * NBUF + j
                mT = m + 2
                mG = m + 1
                bT = (j + 2) % NBUF
                bG = (j + 1) % NBUF

                @pl.when(mT < S)
                def _():
                    @pl.when(mT >= NBUF)
                    def _():
                        s_wait(bT)
                    t_issue(bT)

                @pl.when(mG < S)
                def _():
                    t_wait(bG)
                    g_issue(mG, bG)

                g_wait(j)
                s_issue(m, j)
            return carry

        lax.fori_loop(0, S // NBUF, outer, 0)
        for j in range(NBUF):
            s_wait(j)

    out = run(x_flat, token_emb, pos_emb)
    return out.reshape(B, L, D)


# speedup vs baseline: 5.1775x; 1.0010x over previous
import functools

import jax
import jax.numpy as jnp
from jax import lax
from jax.experimental import pallas as pl
from jax.experimental.pallas import tpu as pltpu
from jax.experimental.pallas import tpu_sc as plsc

NBUF = 8


def kernel(x, token_emb, pos_emb):
    B, L = x.shape
    V, D = token_emb.shape
    info = plsc.get_sparse_core_info()
    NC, NS = info.num_cores, info.num_subcores
    NW = NC * NS
    S = B // NW
    g1 = 128
    g2 = L - g1

    mesh = plsc.VectorSubcoreMesh(core_axis_name="c", subcore_axis_name="s")

    @functools.partial(
        pl.kernel,
        mesh=mesh,
        out_type=jax.ShapeDtypeStruct((B, L, D), jnp.float32),
        compiler_params=pltpu.CompilerParams(use_tc_tiling_on_sc=False),
        scratch_types=[
            pltpu.VMEM((S, L), jnp.int32),
            pltpu.VMEM_SHARED((L, D), jnp.float32),
            pltpu.VMEM((NBUF, L, D), jnp.float32),
            pltpu.SemaphoreType.DMA((NBUF,)),
            pltpu.SemaphoreType.DMA((NBUF,)),
            pltpu.SemaphoreType.DMA((NBUF,)),
        ],
    )
    def run(x_hbm, tok_hbm, pos_hbm, out_hbm, idx_v, pos_sh, rows_v,
            tsem, gsem, ssem):
        sid = lax.axis_index("s")
        wid = sid * NC + lax.axis_index("c")
        seq0 = wid * S
        pltpu.sync_copy(x_hbm.at[pl.ds(seq0, S)], idx_v)

        @pl.when(sid == 0)
        def _():
            pltpu.sync_copy(pos_hbm, pos_sh)

        plsc.subcore_barrier()

        def t_issue(b):
            pltpu.async_copy(pos_sh, rows_v.at[b], tsem.at[b])

        def t_wait(b):
            pltpu.make_async_copy(pos_sh, rows_v.at[b], tsem.at[b]).wait()

        def g_issue(m, b):
            pltpu.async_copy(
                tok_hbm.at[idx_v.at[m, pl.ds(0, g1)]],
                rows_v.at[b, pl.ds(0, g1)], gsem.at[b], add=True)
            pltpu.async_copy(
                tok_hbm.at[idx_v.at[m, pl.ds(g1, g2)]],
                rows_v.at[b, pl.ds(g1, g2)], gsem.at[b], add=True)

        def g_wait(b):
            pltpu.make_async_copy(out_hbm.at[seq0], rows_v.at[b],
                                  gsem.at[b]).wait()

        def s_issue(m, b):
            pltpu.async_copy(rows_v.at[b], out_hbm.at[seq0 + m], ssem.at[b])

        def s_wait(b):
            pltpu.make_async_copy(rows_v.at[b], out_hbm.at[seq0],
                                  ssem.at[b]).wait()

        t_issue(0)
        t_issue(1)
        t_wait(0)
        g_issue(0, 0)

        def outer(s0, carry):
            for j in range(NBUF):
                m = s0 * NBUF + j
                mT = m + 2
                mG = m + 1
                bT = (j + 2) % NBUF
                bG = (j + 1) % NBUF

                @pl.when(mT < S)
                def _():
                    @pl.when(mT >= NBUF)
                    def _():
                        s_wait(bT)
                    t_issue(bT)

                @pl.when(mG < S)
                def _():
                    t_wait(bG)
                    g_issue(mG, bG)

                g_wait(j)
                s_issue(m, j)
            return carry

        lax.fori_loop(0, S // NBUF, outer, 0)
        for j in range(NBUF):
            s_wait(j)

    return run(x, token_emb, pos_emb)
